# baseline (speedup 1.0000x reference)
import functools

import jax
import jax.numpy as jnp
from jax import lax
from jax.experimental import pallas as pl
from jax.experimental.pallas import tpu as pltpu
from jax.experimental.pallas import tpu_sc as plsc


@functools.cache
def _sc_fused(E, D, G):
    NC, NS = 2, 16
    NW = NC * NS
    EPW = E // NW
    SUB = 80
    NBUF = 5
    NSUB = EPW // SUB
    BPC = SUB // 16
    GD = G * D
    KD = D // 16

    mesh = plsc.VectorSubcoreMesh(core_axis_name="c", subcore_axis_name="s")

    @functools.partial(
        pl.kernel,
        mesh=mesh,
        out_type=(
            jax.ShapeDtypeStruct((NW, GD), jnp.float32),
            jax.ShapeDtypeStruct((E,), jnp.float32),
        ),
        scratch_types=[
            pltpu.VMEM((NBUF * SUB * D,), jnp.float32),
            pltpu.VMEM((NBUF * SUB,), jnp.int32),
            pltpu.VMEM((EPW,), jnp.float32),
            pltpu.VMEM((GD,), jnp.float32),
            pltpu.VMEM((D,), jnp.float32),
            pltpu.VMEM((16,), jnp.float32),
            pltpu.SemaphoreType.DMA,
            pltpu.SemaphoreType.DMA,
            pltpu.SemaphoreType.DMA,
            pltpu.SemaphoreType.DMA,
            pltpu.SemaphoreType.DMA,
        ],
    )
    def body(x_hbm, seg_hbm, w_hbm, b_hbm, outp_hbm, outw_hbm,
             xbuf, segb, wfull, acc, wvec, bvec, s0, s1, s2, s3, s4):
        sems = (s0, s1, s2, s3, s4)
        wid = lax.axis_index("c") * NS + lax.axis_index("s")
        base = wid * EPW
        z16 = jnp.zeros((16,), jnp.float32)
        lane = [jnp.full((16,), j, jnp.int32) for j in range(16)]
        iota = lax.iota(jnp.int32, 16)

        fold_pat, merge_a, merge_b, merge_m = {}, {}, {}, {}
        for L in (16, 8, 4, 2):
            n = 16 // L
            Lh = L // 2
            sh = Lh.bit_length() - 1
            fold_pat[L] = (iota & ~(L - 1)) + ((iota + Lh) & (L - 1))
            s = iota >> sh
            o = iota & (Lh - 1)
            merge_a[L] = ((s * L) + o) & 15
            merge_b[L] = (((s - n) * L) + o) & 15
            merge_m[L] = s < n

        def perm(v, pat):
            return jnp.take_along_axis(v, pat, axis=0)

        pltpu.sync_copy(w_hbm, wvec)
        pltpu.sync_copy(b_hbm, bvec)
        wreg = [wvec[pl.ds(k * 16, 16)] for k in range(KD)]
        breg = bvec[pl.ds(0, 16)]

        def zero_body(i, c):
            acc[pl.ds(i * 16, 16)] = z16
            return c

        lax.fori_loop(0, GD // 16, zero_body, 0)

        def issue(ci, b):
            cb = base + ci * SUB
            pltpu.async_copy(x_hbm.at[pl.ds(cb * D, SUB * D)],
                             xbuf.at[pl.ds(b * SUB * D, SUB * D)], sems[b])
            pltpu.async_copy(seg_hbm.at[pl.ds(cb, SUB)],
                             segb.at[pl.ds(b * SUB, SUB)], sems[b])

        def drain(b):
            pltpu.make_async_copy(x_hbm.at[pl.ds(0, SUB * D)],
                                  xbuf.at[pl.ds(b * SUB * D, SUB * D)],
                                  sems[b]).wait()
            pltpu.make_async_copy(seg_hbm.at[pl.ds(0, SUB)],
                                  segb.at[pl.ds(b * SUB, SUB)],
                                  sems[b]).wait()

        for b in range(NBUF):
            issue(b, b)

        def chunk(ci, c):
            slot = ci % NBUF
            for k in range(NBUF):
                @pl.when(slot == k)
                def _(k=k):
                    drain(k)

            def blk(bi, c2):
                e0 = slot * SUB + bi * 16
                ge0 = ci * SUB + bi * 16
                xb = e0 * D

                parts = []
                for j in range(16):
                    eoff = xb + j * D
                    m = [xbuf[pl.ds(eoff + k * 16, 16)] * wreg[k]
                         for k in range(KD)]
                    m = [m[0] + m[1], m[2] + m[3], m[4] + m[5], m[6] + m[7]]
                    parts.append((m[0] + m[1]) + (m[2] + m[3]))

                items = [(p, 16) for p in parts]
                while len(items) > 1:
                    nxt = []
                    for i in range(0, len(items), 2):
                        (a, L) = items[i]
                        (bv, _) = items[i + 1]
                        fa = a + perm(a, fold_pat[L])
                        fb = bv + perm(bv, fold_pat[L])
                        cv = jnp.where(merge_m[L],
                                       perm(fa, merge_a[L]),
                                       perm(fb, merge_b[L]))
                        nxt.append((cv, L // 2))
                    items = nxt
                sacc = items[0][0] + breg

                ex = jnp.exp(sacc * 2.0)
                wv = 1.0 - 2.0 / (ex + 1.0)
                wfull[pl.ds(ge0, 16)] = wv

                segv = jnp.minimum(segb[pl.ds(e0, 16)], G - 1)
                s_first = segv[0]
                s_last = segv[15]

                @pl.when(s_first == s_last)
                def _():
                    accs = [z16 for _ in range(KD)]
                    for j in range(16):
                        wjv = jnp.take_along_axis(wv, lane[j], axis=0)
                        eoff = xb + j * D
                        for k in range(KD):
                            accs[k] = accs[k] + \
                                xbuf[pl.ds(eoff + k * 16, 16)] * wjv
                    offb = s_first * D
                    for k in range(KD):
                        plsc.addupdate(acc.at[pl.ds(offb + k * 16, 16)],
                                       accs[k])

                @pl.when(s_first != s_last)
                def _():
                    offv = segv * D
                    for j in range(16):
                        off = offv[j]
                        wj = wv[j]
                        eoff = xb + j * D
                        for k in range(KD):
                            xk = xbuf[pl.ds(eoff + k * 16, 16)]
                            plsc.addupdate(
                                acc.at[pl.ds(off + k * 16, 16)],
                                xk * wj)
                return c2

            lax.fori_loop(0, BPC, blk, 0, unroll=2)

            nci = ci + NBUF

            @pl.when(nci < NSUB)
            def _():
                for k in range(NBUF):
                    @pl.when(slot == k)
                    def _(k=k):
                        issue(nci, k)

            return c

        lax.fori_loop(0, NSUB, chunk, 0)
        pltpu.sync_copy(wfull, outw_hbm.at[pl.ds(base, EPW)])
        pltpu.sync_copy(acc, outp_hbm.at[wid])

    return body


def _reduce_partials(p, G, D):
    def body(p_ref, o_ref):
        o_ref[...] = jnp.sum(p_ref[...], axis=0, keepdims=True)

    return pl.pallas_call(
        body,
        out_shape=jax.ShapeDtypeStruct((1, G * D), jnp.float32),
    )(p)


def kernel(edge_feats, segment_ids, num_segments, W, b):
    E, D = edge_feats.shape
    G = 256
    b16 = jnp.broadcast_to(b.reshape(1), (16,)).astype(jnp.float32)
    partials, wflat = _sc_fused(E, D, G)(
        edge_feats.reshape(-1), segment_ids, W.reshape(-1), b16
    )
    h_g_sum = _reduce_partials(partials, G, D).reshape(G, D)
    return (h_g_sum, wflat.reshape(E, 1))


# speedup vs baseline: 1.2228x; 1.2228x over previous
import functools

import jax
import jax.numpy as jnp
from jax import lax
from jax.experimental import pallas as pl
from jax.experimental.pallas import tpu as pltpu
from jax.experimental.pallas import tpu_sc as plsc


@functools.cache
def _sc_fused(E, D, G):
    NC, NS = 2, 16
    NW = NC * NS
    EPW = E // NW
    SUB = 80
    NBUF = 5
    NSUB = EPW // SUB
    BPC = SUB // 16
    GD = G * D
    KD = D // 16

    mesh = plsc.VectorSubcoreMesh(core_axis_name="c", subcore_axis_name="s")

    @functools.partial(
        pl.kernel,
        mesh=mesh,
        out_type=(
            jax.ShapeDtypeStruct((NW, GD), jnp.float32),
            jax.ShapeDtypeStruct((E,), jnp.float32),
        ),
        scratch_types=[
            pltpu.VMEM((NBUF * SUB * D,), jnp.float32),
            pltpu.VMEM((NBUF * SUB,), jnp.int32),
            pltpu.VMEM((EPW,), jnp.float32),
            pltpu.VMEM((GD,), jnp.float32),
            pltpu.VMEM((D,), jnp.float32),
            pltpu.VMEM((16,), jnp.float32),
            pltpu.SemaphoreType.DMA,
            pltpu.SemaphoreType.DMA,
            pltpu.SemaphoreType.DMA,
            pltpu.SemaphoreType.DMA,
            pltpu.SemaphoreType.DMA,
        ],
    )
    def body(x_hbm, seg_hbm, w_hbm, b_hbm, outp_hbm, outw_hbm,
             xbuf, segb, wfull, acc, wvec, bvec, s0, s1, s2, s3, s4):
        sems = (s0, s1, s2, s3, s4)
        wid = lax.axis_index("c") * NS + lax.axis_index("s")
        base = wid * EPW
        z16 = jnp.zeros((16,), jnp.float32)
        lane = [jnp.full((16,), j, jnp.int32) for j in range(16)]
        iota = lax.iota(jnp.int32, 16)

        fold_pat, merge_a, merge_b, merge_m = {}, {}, {}, {}
        for L in (16, 8, 4, 2):
            n = 16 // L
            Lh = L // 2
            sh = Lh.bit_length() - 1
            fold_pat[L] = (iota & ~(L - 1)) + ((iota + Lh) & (L - 1))
            s = iota >> sh
            o = iota & (Lh - 1)
            merge_a[L] = ((s * L) + o) & 15
            merge_b[L] = (((s - n) * L) + o) & 15
            merge_m[L] = s < n

        def perm(v, pat):
            return jnp.take_along_axis(v, pat, axis=0)

        pltpu.sync_copy(w_hbm, wvec)
        pltpu.sync_copy(b_hbm, bvec)
        wreg = [wvec[pl.ds(k * 16, 16)] for k in range(KD)]
        breg = bvec[pl.ds(0, 16)]

        def zero_body(i, c):
            acc[pl.ds(i * 16, 16)] = z16
            return c

        lax.fori_loop(0, GD // 16, zero_body, 0)

        def issue(ci, b):
            cb = base + ci * SUB
            pltpu.async_copy(x_hbm.at[pl.ds(cb * D, SUB * D)],
                             xbuf.at[pl.ds(b * SUB * D, SUB * D)], sems[b])
            pltpu.async_copy(seg_hbm.at[pl.ds(cb, SUB)],
                             segb.at[pl.ds(b * SUB, SUB)], sems[b])

        def drain(b):
            pltpu.make_async_copy(x_hbm.at[pl.ds(0, SUB * D)],
                                  xbuf.at[pl.ds(b * SUB * D, SUB * D)],
                                  sems[b]).wait()
            pltpu.make_async_copy(seg_hbm.at[pl.ds(0, SUB)],
                                  segb.at[pl.ds(b * SUB, SUB)],
                                  sems[b]).wait()

        for b in range(NBUF):
            issue(b, b)

        def chunk(ci, c):
            slot = ci % NBUF
            for k in range(NBUF):
                @pl.when(slot == k)
                def _(k=k):
                    drain(k)

            @plsc.parallel_loop(0, BPC, 1, unroll=1)
            def blk(bi):
                e0 = slot * SUB + bi * 16
                ge0 = ci * SUB + bi * 16
                xb = e0 * D

                parts = []
                for j in range(16):
                    eoff = xb + j * D
                    m = [xbuf[pl.ds(eoff + k * 16, 16)] * wreg[k]
                         for k in range(KD)]
                    m = [m[0] + m[1], m[2] + m[3], m[4] + m[5], m[6] + m[7]]
                    parts.append((m[0] + m[1]) + (m[2] + m[3]))

                items = [(p, 16) for p in parts]
                while len(items) > 1:
                    nxt = []
                    for i in range(0, len(items), 2):
                        (a, L) = items[i]
                        (bv, _) = items[i + 1]
                        fa = a + perm(a, fold_pat[L])
                        fb = bv + perm(bv, fold_pat[L])
                        cv = jnp.where(merge_m[L],
                                       perm(fa, merge_a[L]),
                                       perm(fb, merge_b[L]))
                        nxt.append((cv, L // 2))
                    items = nxt
                sacc = items[0][0] + breg

                ex = jnp.exp(sacc * 2.0)
                wv = 1.0 - 2.0 / (ex + 1.0)
                wfull[pl.ds(ge0, 16)] = wv

                segv = jnp.minimum(segb[pl.ds(e0, 16)], G - 1)
                s_first = segv[0]
                s_last = segv[15]

                @pl.when(s_first == s_last)
                def _():
                    accs = [z16 for _ in range(KD)]
                    for j in range(16):
                        wjv = jnp.take_along_axis(wv, lane[j], axis=0)
                        eoff = xb + j * D
                        for k in range(KD):
                            accs[k] = accs[k] + \
                                xbuf[pl.ds(eoff + k * 16, 16)] * wjv
                    offb = s_first * D
                    for k in range(KD):
                        plsc.addupdate(acc.at[pl.ds(offb + k * 16, 16)],
                                       accs[k])

                @pl.when(s_first != s_last)
                def _():
                    offv = segv * D
                    for j in range(16):
                        off = offv[j]
                        wj = wv[j]
                        eoff = xb + j * D
                        for k in range(KD):
                            xk = xbuf[pl.ds(eoff + k * 16, 16)]
                            plsc.addupdate(
                                acc.at[pl.ds(off + k * 16, 16)],
                                xk * wj)

            nci = ci + NBUF

            @pl.when(nci < NSUB)
            def _():
                for k in range(NBUF):
                    @pl.when(slot == k)
                    def _(k=k):
                        issue(nci, k)

            return c

        lax.fori_loop(0, NSUB, chunk, 0)
        pltpu.sync_copy(wfull, outw_hbm.at[pl.ds(base, EPW)])
        pltpu.sync_copy(acc, outp_hbm.at[wid])

    return body


def _reduce_partials(p, G, D):
    def body(p_ref, o_ref):
        o_ref[...] = jnp.sum(p_ref[...], axis=0, keepdims=True)

    return pl.pallas_call(
        body,
        out_shape=jax.ShapeDtypeStruct((1, G * D), jnp.float32),
    )(p)


def kernel(edge_feats, segment_ids, num_segments, W, b):
    E, D = edge_feats.shape
    G = 256
    b16 = jnp.broadcast_to(b.reshape(1), (16,)).astype(jnp.float32)
    partials, wflat = _sc_fused(E, D, G)(
        edge_feats.reshape(-1), segment_ids, W.reshape(-1), b16
    )
    h_g_sum = _reduce_partials(partials, G, D).reshape(G, D)
    return (h_g_sum, wflat.reshape(E, 1))


# speedup vs baseline: 1.2829x; 1.0492x over previous
import functools

import jax
import jax.numpy as jnp
from jax import lax
from jax.experimental import pallas as pl
from jax.experimental.pallas import tpu as pltpu
from jax.experimental.pallas import tpu_sc as plsc


@functools.cache
def _sc_fused(E, D, G):
    NC, NS = 2, 16
    NW = NC * NS
    EPW = E // NW
    SUB = 80
    NBUF = 5
    NSUB = EPW // SUB
    BPC = SUB // 16
    GD = G * D
    KD = D // 16

    mesh = plsc.VectorSubcoreMesh(core_axis_name="c", subcore_axis_name="s")

    @functools.partial(
        pl.kernel,
        mesh=mesh,
        out_type=(
            jax.ShapeDtypeStruct((NW, GD), jnp.float32),
            jax.ShapeDtypeStruct((E,), jnp.float32),
        ),
        scratch_types=[
            pltpu.VMEM((NBUF * SUB * D,), jnp.float32),
            pltpu.VMEM((NBUF * SUB,), jnp.int32),
            pltpu.VMEM((EPW,), jnp.float32),
            pltpu.VMEM((GD,), jnp.float32),
            pltpu.VMEM((D,), jnp.float32),
            pltpu.VMEM((16,), jnp.float32),
            pltpu.SemaphoreType.DMA,
            pltpu.SemaphoreType.DMA,
            pltpu.SemaphoreType.DMA,
            pltpu.SemaphoreType.DMA,
            pltpu.SemaphoreType.DMA,
        ],
    )
    def body(x_hbm, seg_hbm, w_hbm, b_hbm, outp_hbm, outw_hbm,
             xbuf, segb, wfull, acc, wvec, bvec, s0, s1, s2, s3, s4):
        sems = (s0, s1, s2, s3, s4)
        wid = lax.axis_index("c") * NS + lax.axis_index("s")
        base = wid * EPW
        z16 = jnp.zeros((16,), jnp.float32)
        lane = [jnp.full((16,), j, jnp.int32) for j in range(16)]
        iota = lax.iota(jnp.int32, 16)

        fold_pat, merge_a, merge_b, merge_m = {}, {}, {}, {}
        for L in (16, 8, 4, 2):
            n = 16 // L
            Lh = L // 2
            sh = Lh.bit_length() - 1
            fold_pat[L] = (iota & ~(L - 1)) + ((iota + Lh) & (L - 1))
            s = iota >> sh
            o = iota & (Lh - 1)
            merge_a[L] = ((s * L) + o) & 15
            merge_b[L] = (((s - n) * L) + o) & 15
            merge_m[L] = s < n

        def perm(v, pat):
            return jnp.take_along_axis(v, pat, axis=0)

        pltpu.sync_copy(w_hbm, wvec)
        pltpu.sync_copy(b_hbm, bvec)
        wreg = [wvec[pl.ds(k * 16, 16)] for k in range(KD)]
        breg = bvec[pl.ds(0, 16)]

        def zero_body(i, c):
            acc[pl.ds(i * 16, 16)] = z16
            return c

        lax.fori_loop(0, GD // 16, zero_body, 0)

        def issue(ci, b):
            cb = base + ci * SUB
            pltpu.async_copy(x_hbm.at[pl.ds(cb * D, SUB * D)],
                             xbuf.at[pl.ds(b * SUB * D, SUB * D)], sems[b])
            pltpu.async_copy(seg_hbm.at[pl.ds(cb, SUB)],
                             segb.at[pl.ds(b * SUB, SUB)], sems[b])

        def drain(b):
            pltpu.make_async_copy(x_hbm.at[pl.ds(0, SUB * D)],
                                  xbuf.at[pl.ds(b * SUB * D, SUB * D)],
                                  sems[b]).wait()
            pltpu.make_async_copy(seg_hbm.at[pl.ds(0, SUB)],
                                  segb.at[pl.ds(b * SUB, SUB)],
                                  sems[b]).wait()

        for b in range(NBUF):
            issue(b, b)

        def chunk(ci, c):
            slot = ci % NBUF
            for k in range(NBUF):
                @pl.when(slot == k)
                def _(k=k):
                    drain(k)

            @plsc.parallel_loop(0, BPC, 1, unroll=1)
            def blk(bi):
                e0 = slot * SUB + bi * 16
                ge0 = ci * SUB + bi * 16
                xb = e0 * D

                stack = []
                for j in range(16):
                    eoff = xb + j * D
                    m = [xbuf[pl.ds(eoff + k * 16, 16)] * wreg[k]
                         for k in range(KD)]
                    m = [m[0] + m[1], m[2] + m[3], m[4] + m[5], m[6] + m[7]]
                    v = (m[0] + m[1]) + (m[2] + m[3])
                    L = 16
                    while stack and stack[-1][1] == L:
                        a, _ = stack.pop()
                        fa = a + perm(a, fold_pat[L])
                        fb = v + perm(v, fold_pat[L])
                        v = jnp.where(merge_m[L],
                                      perm(fa, merge_a[L]),
                                      perm(fb, merge_b[L]))
                        L //= 2
                    stack.append((v, L))
                sacc = stack[0][0] + breg

                ex = jnp.exp(sacc * 2.0)
                wv = 1.0 - 2.0 / (ex + 1.0)
                wfull[pl.ds(ge0, 16)] = wv

                segv = jnp.minimum(segb[pl.ds(e0, 16)], G - 1)
                s_first = segv[0]
                s_last = segv[15]

                @pl.when(s_first == s_last)
                def _():
                    accs = [z16 for _ in range(KD)]
                    for j in range(16):
                        wjv = jnp.take_along_axis(wv, lane[j], axis=0)
                        eoff = xb + j * D
                        for k in range(KD):
                            accs[k] = accs[k] + \
                                xbuf[pl.ds(eoff + k * 16, 16)] * wjv
                    offb = s_first * D
                    for k in range(KD):
                        plsc.addupdate(acc.at[pl.ds(offb + k * 16, 16)],
                                       accs[k])

                @pl.when(s_first != s_last)
                def _():
                    offv = segv * D
                    for j in range(16):
                        off = offv[j]
                        wj = wv[j]
                        eoff = xb + j * D
                        for k in range(KD):
                            xk = xbuf[pl.ds(eoff + k * 16, 16)]
                            plsc.addupdate(
                                acc.at[pl.ds(off + k * 16, 16)],
                                xk * wj)

            nci = ci + NBUF

            @pl.when(nci < NSUB)
            def _():
                for k in range(NBUF):
                    @pl.when(slot == k)
                    def _(k=k):
                        issue(nci, k)

            return c

        lax.fori_loop(0, NSUB, chunk, 0)
        pltpu.sync_copy(wfull, outw_hbm.at[pl.ds(base, EPW)])
        pltpu.sync_copy(acc, outp_hbm.at[wid])

    return body


def _reduce_partials(p, G, D):
    def body(p_ref, o_ref):
        o_ref[...] = jnp.sum(p_ref[...], axis=0, keepdims=True)

    return pl.pallas_call(
        body,
        out_shape=jax.ShapeDtypeStruct((1, G * D), jnp.float32),
    )(p)


def kernel(edge_feats, segment_ids, num_segments, W, b):
    E, D = edge_feats.shape
    G = 256
    b16 = jnp.broadcast_to(b.reshape(1), (16,)).astype(jnp.float32)
    partials, wflat = _sc_fused(E, D, G)(
        edge_feats.reshape(-1), segment_ids, W.reshape(-1), b16
    )
    h_g_sum = _reduce_partials(partials, G, D).reshape(G, D)
    return (h_g_sum, wflat.reshape(E, 1))
